# Initial kernel scaffold; baseline (speedup 1.0000x reference)
import jax
import jax.numpy as jnp
from jax.experimental import pallas as pl


def kernel(x, edge_index, W0, b0, g0, be0, W1, b1, g1, be1, W2, b2, g2, be2, Wl, bl):
    raise NotImplementedError("write your pallas kernel here")


# speedup vs baseline: 17.0444x; 17.0444x over previous
import functools

import jax
import jax.numpy as jnp
from jax import lax
from jax.experimental import pallas as pl
from jax.experimental.pallas import tpu as pltpu
from jax.experimental.pallas import tpu_sc as plsc

N = 10000
E = 320000
F = 128
H = 128
HH = H // 2
C = 40

NC = 2
NS = 16
NW = NC * NS
EW = E // NW
ET = E // NS
K = 100
ECH = ET // K
NPAD = 10240
RPT = NPAD // NS

_mesh = plsc.VectorSubcoreMesh(core_axis_name="c", subcore_axis_name="s")


DW = 8
DCH = EW // K


@functools.partial(
    pl.kernel,
    out_type=jax.ShapeDtypeStruct((NC, NPAD, DW), jnp.float32),
    mesh=_mesh,
    scratch_types=[
        pltpu.VMEM((DCH, K), jnp.int32),
        pltpu.VMEM((K, DW), jnp.float32),
        pltpu.VMEM_SHARED((NPAD, DW), jnp.float32),
    ],
    compiler_params=pltpu.CompilerParams(use_tc_tiling_on_sc=False),
)
def _deg_kernel(dst_hbm, ones_hbm, zeros_hbm, out_hbm, dst_v, ones_v, acc):
    c = lax.axis_index("c")
    s = lax.axis_index("s")
    w = s * NC + c
    pltpu.sync_copy(dst_hbm.at[w], dst_v)
    pltpu.sync_copy(ones_hbm, ones_v)
    pltpu.sync_copy(zeros_hbm, acc.at[pl.ds(s * RPT, RPT)])
    plsc.subcore_barrier()

    def body(j, _):
        pltpu.sync_copy(ones_v, acc.at[dst_v.at[j]], add=True)
        return ()
    lax.fori_loop(0, DCH, body, ())

    plsc.subcore_barrier()
    pltpu.sync_copy(acc.at[pl.ds(s * RPT, RPT)],
                    out_hbm.at[c, pl.ds(s * RPT, RPT)])


@functools.partial(
    pl.kernel,
    out_type=jax.ShapeDtypeStruct((NC, NPAD, HH), jnp.float32),
    mesh=_mesh,
    scratch_types=[
        pltpu.VMEM((ECH, K), jnp.int32),
        pltpu.VMEM((ECH, K), jnp.int32),
        pltpu.VMEM((K, HH), jnp.float32),
        pltpu.VMEM((K, HH), jnp.float32),
        pltpu.VMEM_SHARED((NPAD, HH), jnp.float32),
        pltpu.SemaphoreType.DMA,
        pltpu.SemaphoreType.DMA,
    ],
    compiler_params=pltpu.CompilerParams(use_tc_tiling_on_sc=False),
)
def _agg_kernel(t_hbm, src_hbm, dst_hbm, zeros_hbm, out_hbm,
                src_v, dst_v, rows0, rows1, acc, sem0, sem1):
    c = lax.axis_index("c")
    s = lax.axis_index("s")
    pltpu.sync_copy(src_hbm.at[s], src_v)
    pltpu.sync_copy(dst_hbm.at[s], dst_v)
    pltpu.sync_copy(zeros_hbm, acc.at[pl.ds(s * RPT, RPT)])
    plsc.subcore_barrier()

    th = t_hbm.at[c]

    def body(i, _):
        j0 = 2 * i
        j1 = 2 * i + 1
        g0 = pltpu.async_copy(th.at[src_v.at[j0]], rows0, sem0)
        g1 = pltpu.async_copy(th.at[src_v.at[j1]], rows1, sem1)
        g0.wait()
        pltpu.sync_copy(rows0, acc.at[dst_v.at[j0]], add=True)
        g1.wait()
        pltpu.sync_copy(rows1, acc.at[dst_v.at[j1]], add=True)
        return ()
    lax.fori_loop(0, ECH // 2, body, ())

    plsc.subcore_barrier()
    pltpu.sync_copy(acc.at[pl.ds(s * RPT, RPT)],
                    out_hbm.at[c, pl.ds(s * RPT, RPT)])


BR = 1000


def _dinv_of(deg_blk):
    return lax.rsqrt(deg_blk[0, :, 0:1] + deg_blk[1, :, 0:1] + 1.0)


def _split_store(o_ref, t):
    o_ref[0] = t[:, :HH]
    o_ref[1] = t[:, HH:]


def _tc0_body(x_ref, w_ref, deg_ref, o_ref):
    dinv = _dinv_of(deg_ref[...])
    t = jnp.dot(x_ref[...], w_ref[...],
                preferred_element_type=jnp.float32) * dinv
    _split_store(o_ref, t)


def _ln_relu(z, g, be):
    mu = jnp.mean(z, axis=-1, keepdims=True)
    zc = z - mu
    var = jnp.mean(zc * zc, axis=-1, keepdims=True)
    h = zc * lax.rsqrt(var + 1e-5) * g + be
    return jnp.maximum(h, 0.0)


def _pre(a_ref, t_ref, deg_ref, b_ref):
    dinv = _dinv_of(deg_ref[...])
    z = jnp.concatenate(
        [a_ref[0] + t_ref[0], a_ref[1] + t_ref[1]], axis=-1)
    return dinv, dinv * z + b_ref[...]


def _tcmid_body(a_ref, t_ref, deg_ref, b_ref, g_ref, be_ref, w_ref, o_ref):
    dinv, z = _pre(a_ref, t_ref, deg_ref, b_ref)
    h = _ln_relu(z, g_ref[...], be_ref[...])
    t = jnp.dot(h, w_ref[...], preferred_element_type=jnp.float32) * dinv
    _split_store(o_ref, t)


def _tcfin_body(a_ref, t_ref, deg_ref, b_ref, g_ref, be_ref, wl_ref, bl_ref,
                o_ref):
    _, z = _pre(a_ref, t_ref, deg_ref, b_ref)
    h = _ln_relu(z, g_ref[...], be_ref[...])
    o_ref[...] = jnp.dot(h, wl_ref[...],
                         preferred_element_type=jnp.float32) + bl_ref[...]


def _row_spec(width):
    return pl.BlockSpec((BR, width), lambda i: (i, 0))


def _half_spec():
    return pl.BlockSpec((2, BR, HH), lambda i: (0, i, 0))


def _deg_spec():
    return pl.BlockSpec((2, BR, DW), lambda i: (0, i, 0))


def _full_spec(shape):
    return pl.BlockSpec(shape, lambda i: tuple(0 for _ in shape))


_SPLIT_OUT = jax.ShapeDtypeStruct((2, N, HH), jnp.float32)


def _tc0(x, w, deg_t):
    return pl.pallas_call(
        _tc0_body,
        grid=(N // BR,),
        in_specs=[_row_spec(F), _full_spec((F, H)), _deg_spec()],
        out_specs=_half_spec(),
        out_shape=_SPLIT_OUT,
    )(x, w, deg_t)


def _tcmid(a, t, deg_t, b, g, be, w):
    return pl.pallas_call(
        _tcmid_body,
        grid=(N // BR,),
        in_specs=[_half_spec(), _half_spec(), _deg_spec(),
                  _full_spec((1, H)), _full_spec((1, H)), _full_spec((1, H)),
                  _full_spec((H, H))],
        out_specs=_half_spec(),
        out_shape=_SPLIT_OUT,
    )(a, t, deg_t, b, g, be, w)


def _tcfin(a, t, deg_t, b, g, be, wl, bl):
    return pl.pallas_call(
        _tcfin_body,
        grid=(N // BR,),
        in_specs=[_half_spec(), _half_spec(), _deg_spec(),
                  _full_spec((1, H)), _full_spec((1, H)), _full_spec((1, H)),
                  _full_spec((H, C)), _full_spec((1, C))],
        out_specs=_row_spec(C),
        out_shape=jax.ShapeDtypeStruct((N, C), jnp.float32),
    )(a, t, deg_t, b, g, be, wl, bl)


def kernel(x, edge_index, W0, b0, g0, be0, W1, b1, g1, be1, W2, b2, g2, be2,
           Wl, bl):
    src = edge_index[0].reshape(NS, ECH, K)
    dst = edge_index[1].reshape(NS, ECH, K)
    dst_flat = edge_index[1].reshape(NW, DCH, K)

    ones8 = jnp.ones((K, DW), jnp.float32)
    zeros8 = jnp.zeros((RPT, DW), jnp.float32)
    deg_t = _deg_kernel(dst_flat, ones8, zeros8)
    zeros = jnp.zeros((RPT, HH), jnp.float32)

    b0r, g0r, be0r = b0.reshape(1, H), g0.reshape(1, H), be0.reshape(1, H)
    b1r, g1r, be1r = b1.reshape(1, H), g1.reshape(1, H), be1.reshape(1, H)
    b2r, g2r, be2r = b2.reshape(1, H), g2.reshape(1, H), be2.reshape(1, H)
    blr = bl.reshape(1, C)

    t0 = _tc0(x, W0, deg_t)
    a0 = _agg_kernel(t0, src, dst, zeros)
    t1 = _tcmid(a0, t0, deg_t, b0r, g0r, be0r, W1)
    a1 = _agg_kernel(t1, src, dst, zeros)
    t2 = _tcmid(a1, t1, deg_t, b1r, g1r, be1r, W2)
    a2 = _agg_kernel(t2, src, dst, zeros)
    return _tcfin(a2, t2, deg_t, b2r, g2r, be2r, Wl, blr)
